# baseline (speedup 1.0000x reference)
import jax
import jax.numpy as jnp
from jax.experimental import pallas as pl
from jax.experimental.pallas import tpu as pltpu


def _pick_block(n, target):
    c = target
    while c > 8:
        if n % c == 0 and c % 8 == 0:
            return c
        c //= 2
    return n


def _body(adj_ref, x_ref, w1t_ref, b1_ref, w2t_ref, b2_ref,
          wp1t_ref, bp1_ref, wp2t_ref, bp2_ref,
          z_ref, emb_ref, p1_ref, h2_ref):
    p = pl.program_id(0)
    i = pl.program_id(1)
    bm = adj_ref.shape[0]

    @pl.when((p == 0) & (i == 0))
    def _compute_p1():
        p1_ref[...] = (
            jnp.dot(x_ref[...], w1t_ref[...], preferred_element_type=jnp.float32)
            + b1_ref[...]
        )

    @pl.when(p == 0)
    def _phase_a():
        h1 = jnp.maximum(
            jnp.dot(adj_ref[...], p1_ref[...],
                    preferred_element_type=jnp.float32),
            0.0)
        h2_ref[pl.ds(i * bm, bm), :] = (
            jnp.dot(h1, w2t_ref[...], preferred_element_type=jnp.float32)
            + b2_ref[...]
        )

    @pl.when(p == 1)
    def _phase_b():
        e = jnp.dot(adj_ref[...], h2_ref[...],
                    preferred_element_type=jnp.float32)
        emb_ref[...] = e
        q = jnp.maximum(
            jnp.dot(e, wp1t_ref[...], preferred_element_type=jnp.float32)
            + bp1_ref[...], 0.0)
        z_ref[...] = (
            jnp.dot(q, wp2t_ref[...], preferred_element_type=jnp.float32)
            + bp2_ref[...]
        )


def kernel(x, Adj_, W1, b1, W2, b2, Wp1, bp1, Wp2, bp2):
    n, in_dim = x.shape
    hid = W1.shape[0]
    emb_d = W2.shape[0]
    proj = Wp2.shape[0]

    bm = _pick_block(n, 200)
    ni = n // bm

    w1t = W1.T
    w2t = W2.T
    wp1t = Wp1.T
    wp2t = Wp2.T
    b1r = b1.reshape(1, hid)
    b2r = b2.reshape(1, emb_d)
    bp1r = bp1.reshape(1, proj)
    bp2r = bp2.reshape(1, proj)

    grid = (2, ni)

    def out_idx(p, i):
        return (jnp.where(p == 1, i, 0), 0)

    def const_spec(shape):
        return pl.BlockSpec(shape, lambda p, i: (0, 0))

    z, emb = pl.pallas_call(
        _body,
        grid=grid,
        in_specs=[
            pl.BlockSpec((bm, n), lambda p, i: (i, 0)),
            const_spec((n, in_dim)),
            const_spec((in_dim, hid)),
            const_spec((1, hid)),
            const_spec((hid, emb_d)),
            const_spec((1, emb_d)),
            const_spec((emb_d, proj)),
            const_spec((1, proj)),
            const_spec((proj, proj)),
            const_spec((1, proj)),
        ],
        out_specs=[
            pl.BlockSpec((bm, proj), out_idx),
            pl.BlockSpec((bm, emb_d), out_idx),
        ],
        out_shape=[
            jax.ShapeDtypeStruct((n, proj), jnp.float32),
            jax.ShapeDtypeStruct((n, emb_d), jnp.float32),
        ],
        scratch_shapes=[
            pltpu.VMEM((n, hid), jnp.float32),
            pltpu.VMEM((n, emb_d), jnp.float32),
        ],
        compiler_params=pltpu.CompilerParams(
            dimension_semantics=("arbitrary", "arbitrary"),
        ),
    )(Adj_, x, w1t, b1r, w2t, b2r, wp1t, bp1r, wp2t, bp2r)

    return (z, emb)


# speedup vs baseline: 1.0250x; 1.0250x over previous
import jax
import jax.numpy as jnp
from jax.experimental import pallas as pl
from jax.experimental.pallas import tpu as pltpu


def _pick_block(n, target):
    c = target
    while c > 8:
        if n % c == 0 and c % 8 == 0:
            return c
        c //= 2
    return n


def _body(adj_ref, x_ref, w1t_ref, b1_ref, w2t_ref, b2_ref,
          wp1t_ref, bp1_ref, wp2t_ref, bp2_ref,
          z_ref, emb_ref, p1_ref, h2_ref):
    p = pl.program_id(0)
    i = pl.program_id(1)
    bm = adj_ref.shape[0]

    @pl.when((p == 0) & (i == 0))
    def _compute_p1():
        p1_ref[...] = (
            jnp.dot(x_ref[...], w1t_ref[...], preferred_element_type=jnp.float32)
            + b1_ref[...]
        ).astype(jnp.bfloat16)

    adj_bf = adj_ref[...].astype(jnp.bfloat16)

    @pl.when(p == 0)
    def _phase_a():
        h1 = jnp.maximum(
            jnp.dot(adj_bf, p1_ref[...],
                    preferred_element_type=jnp.float32),
            0.0)
        h2_ref[pl.ds(i * bm, bm), :] = (
            jnp.dot(h1, w2t_ref[...], preferred_element_type=jnp.float32)
            + b2_ref[...]
        ).astype(jnp.bfloat16)

    @pl.when(p == 1)
    def _phase_b():
        e = jnp.dot(adj_bf, h2_ref[...],
                    preferred_element_type=jnp.float32)
        emb_ref[...] = e
        q = jnp.maximum(
            jnp.dot(e, wp1t_ref[...], preferred_element_type=jnp.float32)
            + bp1_ref[...], 0.0)
        z_ref[...] = (
            jnp.dot(q, wp2t_ref[...], preferred_element_type=jnp.float32)
            + bp2_ref[...]
        )


def kernel(x, Adj_, W1, b1, W2, b2, Wp1, bp1, Wp2, bp2):
    n, in_dim = x.shape
    hid = W1.shape[0]
    emb_d = W2.shape[0]
    proj = Wp2.shape[0]

    bm = _pick_block(n, 400)
    ni = n // bm

    w1t = W1.T
    w2t = W2.T
    wp1t = Wp1.T
    wp2t = Wp2.T
    b1r = b1.reshape(1, hid)
    b2r = b2.reshape(1, emb_d)
    bp1r = bp1.reshape(1, proj)
    bp2r = bp2.reshape(1, proj)

    grid = (2, ni)

    def out_idx(p, i):
        return (jnp.where(p == 1, i, 0), 0)

    def const_spec(shape):
        return pl.BlockSpec(shape, lambda p, i: (0, 0))

    z, emb = pl.pallas_call(
        _body,
        grid=grid,
        in_specs=[
            pl.BlockSpec((bm, n), lambda p, i: (i, 0)),
            const_spec((n, in_dim)),
            const_spec((in_dim, hid)),
            const_spec((1, hid)),
            const_spec((hid, emb_d)),
            const_spec((1, emb_d)),
            const_spec((emb_d, proj)),
            const_spec((1, proj)),
            const_spec((proj, proj)),
            const_spec((1, proj)),
        ],
        out_specs=[
            pl.BlockSpec((bm, proj), out_idx),
            pl.BlockSpec((bm, emb_d), out_idx),
        ],
        out_shape=[
            jax.ShapeDtypeStruct((n, proj), jnp.float32),
            jax.ShapeDtypeStruct((n, emb_d), jnp.float32),
        ],
        scratch_shapes=[
            pltpu.VMEM((n, hid), jnp.bfloat16),
            pltpu.VMEM((n, emb_d), jnp.bfloat16),
        ],
        compiler_params=pltpu.CompilerParams(
            dimension_semantics=("arbitrary", "arbitrary"),
            vmem_limit_bytes=128 * 1024 * 1024,
        ),
    )(Adj_, x, w1t, b1r, w2t, b2r, wp1t, bp1r, wp2t, bp2r)

    return (z, emb)


# speedup vs baseline: 1.0444x; 1.0189x over previous
import jax
import jax.numpy as jnp
from jax.experimental import pallas as pl
from jax.experimental.pallas import tpu as pltpu


def _pick_block(n, target):
    c = target
    while c > 8:
        if n % c == 0 and c % 8 == 0:
            return c
        c //= 2
    return n


def _body(adj_ref, x_ref, w1t_ref, b1_ref, w2t_ref, b2_ref,
          wp1t_ref, bp1_ref, wp2t_ref, bp2_ref,
          z_ref, emb_ref, p1_ref, h2_ref):
    p = pl.program_id(0)
    i = pl.program_id(1)
    bm = adj_ref.shape[0]

    @pl.when((p == 0) & (i == 0))
    def _compute_p1():
        p1_ref[...] = (
            jnp.dot(x_ref[...], w1t_ref[...], preferred_element_type=jnp.float32)
            + b1_ref[...]
        )

    @pl.when(p == 0)
    def _phase_a():
        h1 = jnp.maximum(
            jnp.dot(adj_ref[...], p1_ref[...],
                    preferred_element_type=jnp.float32),
            0.0)
        h2_ref[pl.ds(i * bm, bm), :] = (
            jnp.dot(h1, w2t_ref[...], preferred_element_type=jnp.float32)
            + b2_ref[...]
        )

    @pl.when(p == 1)
    def _phase_b():
        e = jnp.dot(adj_ref[...], h2_ref[...],
                    preferred_element_type=jnp.float32)
        emb_ref[...] = e
        q = jnp.maximum(
            jnp.dot(e, wp1t_ref[...], preferred_element_type=jnp.float32)
            + bp1_ref[...], 0.0)
        z_ref[...] = (
            jnp.dot(q, wp2t_ref[...], preferred_element_type=jnp.float32)
            + bp2_ref[...]
        )


def kernel(x, Adj_, W1, b1, W2, b2, Wp1, bp1, Wp2, bp2):
    n, in_dim = x.shape
    hid = W1.shape[0]
    emb_d = W2.shape[0]
    proj = Wp2.shape[0]

    bm = _pick_block(n, 400)
    ni = n // bm

    w1t = W1.T
    w2t = W2.T
    wp1t = Wp1.T
    wp2t = Wp2.T
    b1r = b1.reshape(1, hid)
    b2r = b2.reshape(1, emb_d)
    bp1r = bp1.reshape(1, proj)
    bp2r = bp2.reshape(1, proj)

    grid = (2, ni)

    def out_idx(p, i):
        return (jnp.where(p == 1, i, 0), 0)

    def const_spec(shape):
        return pl.BlockSpec(shape, lambda p, i: (0, 0))

    z, emb = pl.pallas_call(
        _body,
        grid=grid,
        in_specs=[
            pl.BlockSpec((bm, n), lambda p, i: (i, 0)),
            const_spec((n, in_dim)),
            const_spec((in_dim, hid)),
            const_spec((1, hid)),
            const_spec((hid, emb_d)),
            const_spec((1, emb_d)),
            const_spec((emb_d, proj)),
            const_spec((1, proj)),
            const_spec((proj, proj)),
            const_spec((1, proj)),
        ],
        out_specs=[
            pl.BlockSpec((bm, proj), out_idx),
            pl.BlockSpec((bm, emb_d), out_idx),
        ],
        out_shape=[
            jax.ShapeDtypeStruct((n, proj), jnp.float32),
            jax.ShapeDtypeStruct((n, emb_d), jnp.float32),
        ],
        scratch_shapes=[
            pltpu.VMEM((n, hid), jnp.float32),
            pltpu.VMEM((n, emb_d), jnp.float32),
        ],
        compiler_params=pltpu.CompilerParams(
            dimension_semantics=("arbitrary", "arbitrary"),
            vmem_limit_bytes=128 * 1024 * 1024,
        ),
    )(Adj_, x, w1t, b1r, w2t, b2r, wp1t, bp1r, wp2t, bp2r)

    return (z, emb)
